# Initial kernel scaffold; baseline (speedup 1.0000x reference)
import jax
import jax.numpy as jnp
from jax.experimental import pallas as pl


def kernel(vertices, params):
    raise NotImplementedError("write your pallas kernel here")


# speedup vs baseline: 5.7470x; 5.7470x over previous
import functools

import jax
import jax.numpy as jnp
from jax import lax
from jax.experimental import pallas as pl
from jax.experimental.pallas import tpu as pltpu
from jax.experimental.pallas import tpu_sc as plsc

NC, NS, LANES = 2, 16, 16
NW = NC * NS
BS = 2
V = 1024
N = BS * V
K = 16
VPW = N // NW
CH = 16


def _mesh():
    return plsc.VectorSubcoreMesh(
        core_axis_name="c", subcore_axis_name="s", num_cores=NC, num_subcores=NS
    )


def _knn_body(v_ref, idx_ref):
    b = pl.program_id(0)
    vm = v_ref[0]
    inner = lax.dot_general(
        vm, vm, (((1,), (1,)), ((), ())), preferred_element_type=jnp.float32
    )
    q = jnp.sum(vm * vm, axis=1)
    dist = (-2.0 * inner + q[None, :]) + q[:, None]
    col = lax.broadcasted_iota(jnp.int32, (V, V), 1)
    inf = jnp.float32(jnp.inf)
    off = b * V
    outs = []
    for k in range(K + 1):
        m = jnp.min(dist, axis=1)
        cand = jnp.where(dist == m[:, None], col, jnp.int32(2**30))
        amin = jnp.min(cand, axis=1)
        if k > 0:
            outs.append(amin + off)
        dist = jnp.where(col == amin[:, None], inf, dist)
    idx_ref[0] = jnp.stack(outs, axis=-1)


def _knn(vertices):
    return pl.pallas_call(
        _knn_body,
        grid=(BS,),
        in_specs=[pl.BlockSpec((1, V, 3), lambda b: (b, 0, 0))],
        out_specs=pl.BlockSpec((1, V, K), lambda b: (b, 0, 0)),
        out_shape=jax.ShapeDtypeStruct((BS, V, K), jnp.int32),
    )(vertices)


def _sc_dirs(cx, cy, cz, idxf):
    @functools.partial(
        pl.kernel,
        out_type=tuple(jax.ShapeDtypeStruct((N * K,), jnp.float32) for _ in range(3)),
        mesh=_mesh(),
        compiler_params=pltpu.CompilerParams(needs_layout_passes=False, use_tc_tiling_on_sc=False),
        scratch_types=[
            pltpu.VMEM((N,), jnp.float32),
            pltpu.VMEM((N,), jnp.float32),
            pltpu.VMEM((N,), jnp.float32),
            pltpu.VMEM((VPW * K,), jnp.int32),
            pltpu.VMEM((VPW * K,), jnp.float32),
            pltpu.VMEM((VPW * K,), jnp.float32),
            pltpu.VMEM((VPW * K,), jnp.float32),
        ],
    )
    def body(cx_hbm, cy_hbm, cz_hbm, idx_hbm, dx_hbm, dy_hbm, dz_hbm,
             x_v, y_v, z_v, idx_v, dox, doy, doz):
        wid = lax.axis_index("s") * NC + lax.axis_index("c")
        base = wid * VPW
        pltpu.sync_copy(cx_hbm, x_v)
        pltpu.sync_copy(cy_hbm, y_v)
        pltpu.sync_copy(cz_hbm, z_v)
        pltpu.sync_copy(idx_hbm.at[pl.ds(base * K, VPW * K)], idx_v)
        for v in range(VPW):
            iv = idx_v[pl.ds(v * K, K)]
            cidx = jnp.full((LANES,), base + v, jnp.int32)
            for src, dst in ((x_v, dox), (y_v, doy), (z_v, doz)):
                g = plsc.load_gather(src, [iv])
                c = plsc.load_gather(src, [cidx])
                dst[pl.ds(v * K, K)] = g - c
        pltpu.sync_copy(dox, dx_hbm.at[pl.ds(base * K, VPW * K)])
        pltpu.sync_copy(doy, dy_hbm.at[pl.ds(base * K, VPW * K)])
        pltpu.sync_copy(doz, dz_hbm.at[pl.ds(base * K, VPW * K)])

    return body(cx, cy, cz, idxf)


def _norm_body(dx_ref, dy_ref, dz_ref, nx_ref, ny_ref, nz_ref):
    dx, dy, dz = dx_ref[...], dy_ref[...], dz_ref[...]
    n2 = (dx * dx + dy * dy) + dz * dz
    dnm = jnp.maximum(jnp.sqrt(n2), 1e-12)
    nx_ref[...] = dx / dnm
    ny_ref[...] = dy / dnm
    nz_ref[...] = dz / dnm


def _normalize(dx, dy, dz):
    return pl.pallas_call(
        _norm_body,
        out_shape=tuple(jax.ShapeDtypeStruct((N, K), jnp.float32) for _ in range(3)),
    )(dx.reshape(N, K), dy.reshape(N, K), dz.reshape(N, K))


def _mm_body(fm_ref, w_ref, b_ref, dir_ref, fc_ref, fs_ref, sdn_ref, *, oc):
    fo = (
        jnp.dot(fm_ref[...], w_ref[...], preferred_element_type=jnp.float32)
        + b_ref[...]
    )
    fc_ref[...] = fo[:, :oc]
    fs_ref[...] = fo[:, oc:]
    sd = dir_ref[...]
    nrm = jnp.sqrt(jnp.sum(sd * sd, axis=0, keepdims=True))
    sdn_ref[...] = sd / jnp.maximum(nrm, 1e-12)


def _mm(fm, w, b2d, dirs_param):
    ic, oc2 = w.shape
    oc = oc2 // 2
    return pl.pallas_call(
        functools.partial(_mm_body, oc=oc),
        out_shape=(
            jax.ShapeDtypeStruct((N, oc), jnp.float32),
            jax.ShapeDtypeStruct((N, oc), jnp.float32),
            jax.ShapeDtypeStruct((3, oc), jnp.float32),
        ),
    )(fm, w, b2d, dirs_param)


def _sc_agg(fs, fc, dnx_a, dny_a, dnz_a, sx_a, sy_a, sz_a, idxf, oc):
    J = oc // LANES

    @functools.partial(
        pl.kernel,
        out_type=jax.ShapeDtypeStruct((N, oc), jnp.float32),
        mesh=_mesh(),
        compiler_params=pltpu.CompilerParams(needs_layout_passes=False, use_tc_tiling_on_sc=False),
        scratch_types=[
            pltpu.VMEM((oc,), jnp.float32),
            pltpu.VMEM((oc,), jnp.float32),
            pltpu.VMEM((oc,), jnp.float32),
            pltpu.VMEM((VPW * K,), jnp.float32),
            pltpu.VMEM((VPW * K,), jnp.float32),
            pltpu.VMEM((VPW * K,), jnp.float32),
            pltpu.VMEM((VPW * K,), jnp.int32),
            pltpu.VMEM((K, oc), jnp.float32),
            pltpu.VMEM((K, oc), jnp.float32),
            pltpu.VMEM((CH, oc), jnp.float32),
            pltpu.VMEM((CH, oc), jnp.float32),
            pltpu.SemaphoreType.DMA,
            pltpu.SemaphoreType.DMA,
        ],
    )
    def body(fs_hbm, fc_hbm, dnx_hbm, dny_hbm, dnz_hbm, sx_hbm, sy_hbm, sz_hbm,
             idx_hbm, out_hbm,
             sx_v, sy_v, sz_v, dnx_v, dny_v, dnz_v, idx_v,
             rows0, rows1, fc_v, out_v, sem0, sem1):
        wid = lax.axis_index("s") * NC + lax.axis_index("c")
        base = wid * VPW
        pltpu.sync_copy(sx_hbm, sx_v)
        pltpu.sync_copy(sy_hbm, sy_v)
        pltpu.sync_copy(sz_hbm, sz_v)
        pltpu.sync_copy(idx_hbm.at[pl.ds(base * K, VPW * K)], idx_v)
        pltpu.sync_copy(dnx_hbm.at[pl.ds(base * K, VPW * K)], dnx_v)
        pltpu.sync_copy(dny_hbm.at[pl.ds(base * K, VPW * K)], dny_v)
        pltpu.sync_copy(dnz_hbm.at[pl.ds(base * K, VPW * K)], dnz_v)
        rows = (rows0, rows1)
        sems = (sem0, sem1)
        pltpu.async_copy(fs_hbm.at[idx_v.at[pl.ds(0, K)]], rows[0], sems[0])
        pltpu.async_copy(fs_hbm.at[idx_v.at[pl.ds(K, K)]], rows[1], sems[1])

        for ch in range(VPW // CH):
            pltpu.sync_copy(fc_hbm.at[pl.ds(base + ch * CH, CH)], fc_v)

            def pair_body(p, carry, *, ch=ch):
                for bsel in range(2):
                    v = ch * CH + 2 * p + bsel
                    row = rows[bsel]
                    sem = sems[bsel]
                    pltpu.make_async_copy(
                        fs_hbm.at[idx_v.at[pl.ds(v * K, K)]], row, sem
                    ).wait()
                    vi = 2 * p + bsel
                    dxv = dnx_v[pl.ds(v * K, K)]
                    dyv = dny_v[pl.ds(v * K, K)]
                    dzv = dnz_v[pl.ds(v * K, K)]
                    dnx = [dxv[n] for n in range(K)]
                    dny = [dyv[n] for n in range(K)]
                    dnz = [dzv[n] for n in range(K)]

                    def jbody(j, c, *, row=row, vi=vi, dnx=dnx, dny=dny, dnz=dnz):
                        sl = pl.ds(j * LANES, LANES)
                        sx = sx_v[sl]
                        sy = sy_v[sl]
                        sz = sz_v[sl]
                        acc = None
                        for n in range(K):
                            th = jnp.maximum(
                                dnx[n] * sx + dny[n] * sy + dnz[n] * sz, 0.0
                            ) * row[n, sl]
                            acc = th if acc is None else jnp.maximum(acc, th)
                        out_v[vi, sl] = fc_v[vi, sl] + acc
                        return c

                    lax.fori_loop(0, J, jbody, 0)
                    @pl.when(v + 2 < VPW)
                    def _():
                        pltpu.async_copy(
                            fs_hbm.at[idx_v.at[pl.ds((v + 2) * K, K)]], row, sem
                        )
                return carry

            lax.fori_loop(0, CH // 2, pair_body, 0)
            pltpu.sync_copy(out_v, out_hbm.at[pl.ds(base + ch * CH, CH)])

    return body(fs, fc, dnx_a, dny_a, dnz_a, sx_a, sy_a, sz_a, idxf)


def _head_body(fm_ref, w1_ref, b1_ref, g_ref, bt_ref, w2_ref, b2_ref, out_ref):
    fg = jnp.max(fm_ref[...], axis=1)
    h = jnp.dot(fg, w1_ref[...], preferred_element_type=jnp.float32) + b1_ref[...]
    h = h / jnp.sqrt(jnp.float32(1.0 + 1e-5)) * g_ref[...] + bt_ref[...]
    h = jnp.maximum(h, 0.0)
    out_ref[...] = (
        jnp.dot(h, w2_ref[...], preferred_element_type=jnp.float32) + b2_ref[...]
    )


def _head(fm3, cls):
    c = fm3.shape[-1]
    return pl.pallas_call(
        _head_body,
        out_shape=jax.ShapeDtypeStruct((BS, 2), jnp.float32),
    )(
        fm3,
        cls["w1"],
        cls["b1"].reshape(1, -1),
        cls["gamma"].reshape(1, -1),
        cls["beta"].reshape(1, -1),
        cls["w2"],
        cls["b2"].reshape(1, -1),
    )


def kernel(vertices, params):
    convs = params["convs"]
    cls = params["cls"]
    verts = vertices.reshape(N, 3)
    cx, cy, cz = verts[:, 0], verts[:, 1], verts[:, 2]
    idxf = _knn(vertices).reshape(N * K)
    dx, dy, dz = _sc_dirs(cx, cy, cz, idxf)
    dnx, dny, dnz = _normalize(dx, dy, dz)
    dnx, dny, dnz = dnx.reshape(N * K), dny.reshape(N * K), dnz.reshape(N * K)
    fm = verts
    for p in convs:
        oc = p["weights"].shape[1] // 2
        fc, fs, sdn = _mm(fm, p["weights"], p["bias"].reshape(1, -1), p["directions"])
        fm = _sc_agg(fs, fc, dnx, dny, dnz, sdn[0], sdn[1], sdn[2], idxf, oc)
    return _head(fm.reshape(BS, V, -1), cls)


# speedup vs baseline: 6.8588x; 1.1935x over previous
import functools

import jax
import jax.numpy as jnp
from jax import lax
from jax.experimental import pallas as pl
from jax.experimental.pallas import tpu as pltpu
from jax.experimental.pallas import tpu_sc as plsc

NC, NS, LANES = 2, 16, 16
NW = NC * NS
BS = 2
V = 1024
N = BS * V
K = 16
VPW = N // NW
CH = 16


def _mesh():
    return plsc.VectorSubcoreMesh(
        core_axis_name="c", subcore_axis_name="s", num_cores=NC, num_subcores=NS
    )


def _knn_body(v_ref, idx_ref):
    b = pl.program_id(0)
    vm = v_ref[0]
    inner = lax.dot_general(
        vm, vm, (((1,), (1,)), ((), ())), preferred_element_type=jnp.float32
    )
    q = jnp.sum(vm * vm, axis=1)
    dist = (-2.0 * inner + q[None, :]) + q[:, None]
    col = lax.broadcasted_iota(jnp.int32, (V, V), 1)
    inf = jnp.float32(jnp.inf)
    off = b * V
    outs = []
    for k in range(K + 1):
        m = jnp.min(dist, axis=1)
        cand = jnp.where(dist == m[:, None], col, jnp.int32(2**30))
        amin = jnp.min(cand, axis=1)
        if k > 0:
            outs.append(amin + off)
        dist = jnp.where(col == amin[:, None], inf, dist)
    idx_ref[0] = jnp.stack(outs, axis=-1)


def _knn(vertices):
    return pl.pallas_call(
        _knn_body,
        grid=(BS,),
        in_specs=[pl.BlockSpec((1, V, 3), lambda b: (b, 0, 0))],
        out_specs=pl.BlockSpec((1, V, K), lambda b: (b, 0, 0)),
        out_shape=jax.ShapeDtypeStruct((BS, V, K), jnp.int32),
    )(vertices)


def _sc_dirs(cx, cy, cz, idxf):
    @functools.partial(
        pl.kernel,
        out_type=tuple(jax.ShapeDtypeStruct((N * K,), jnp.float32) for _ in range(3)),
        mesh=_mesh(),
        compiler_params=pltpu.CompilerParams(needs_layout_passes=False, use_tc_tiling_on_sc=False),
        scratch_types=[
            pltpu.VMEM((N,), jnp.float32),
            pltpu.VMEM((N,), jnp.float32),
            pltpu.VMEM((N,), jnp.float32),
            pltpu.VMEM((VPW * K,), jnp.int32),
            pltpu.VMEM((VPW * K,), jnp.float32),
            pltpu.VMEM((VPW * K,), jnp.float32),
            pltpu.VMEM((VPW * K,), jnp.float32),
        ],
    )
    def body(cx_hbm, cy_hbm, cz_hbm, idx_hbm, dx_hbm, dy_hbm, dz_hbm,
             x_v, y_v, z_v, idx_v, dox, doy, doz):
        wid = lax.axis_index("s") * NC + lax.axis_index("c")
        base = wid * VPW
        pltpu.sync_copy(cx_hbm, x_v)
        pltpu.sync_copy(cy_hbm, y_v)
        pltpu.sync_copy(cz_hbm, z_v)
        pltpu.sync_copy(idx_hbm.at[pl.ds(base * K, VPW * K)], idx_v)
        for v in range(VPW):
            iv = idx_v[pl.ds(v * K, K)]
            cidx = jnp.full((LANES,), base + v, jnp.int32)
            for src, dst in ((x_v, dox), (y_v, doy), (z_v, doz)):
                g = plsc.load_gather(src, [iv])
                c = plsc.load_gather(src, [cidx])
                dst[pl.ds(v * K, K)] = g - c
        pltpu.sync_copy(dox, dx_hbm.at[pl.ds(base * K, VPW * K)])
        pltpu.sync_copy(doy, dy_hbm.at[pl.ds(base * K, VPW * K)])
        pltpu.sync_copy(doz, dz_hbm.at[pl.ds(base * K, VPW * K)])

    return body(cx, cy, cz, idxf)


def _norm_body(dx_ref, dy_ref, dz_ref, nx_ref, ny_ref, nz_ref):
    dx, dy, dz = dx_ref[...], dy_ref[...], dz_ref[...]
    n2 = (dx * dx + dy * dy) + dz * dz
    dnm = jnp.maximum(jnp.sqrt(n2), 1e-12)
    nx_ref[...] = dx / dnm
    ny_ref[...] = dy / dnm
    nz_ref[...] = dz / dnm


def _normalize(dx, dy, dz):
    return pl.pallas_call(
        _norm_body,
        out_shape=tuple(jax.ShapeDtypeStruct((N, K), jnp.float32) for _ in range(3)),
    )(dx.reshape(N, K), dy.reshape(N, K), dz.reshape(N, K))


def _mm_body(fm_ref, w_ref, b_ref, dir_ref, fc_ref, fs_ref, sdn_ref, *, oc):
    fo = (
        jnp.dot(fm_ref[...], w_ref[...], preferred_element_type=jnp.float32)
        + b_ref[...]
    )
    fc_ref[...] = fo[:, :oc]
    fs_ref[...] = fo[:, oc:]
    sd = dir_ref[...]
    nrm = jnp.sqrt(jnp.sum(sd * sd, axis=0, keepdims=True))
    sdn_ref[...] = sd / jnp.maximum(nrm, 1e-12)


def _mm(fm, w, b2d, dirs_param):
    ic, oc2 = w.shape
    oc = oc2 // 2
    return pl.pallas_call(
        functools.partial(_mm_body, oc=oc),
        out_shape=(
            jax.ShapeDtypeStruct((N, oc), jnp.float32),
            jax.ShapeDtypeStruct((N, oc), jnp.float32),
            jax.ShapeDtypeStruct((3, oc), jnp.float32),
        ),
    )(fm, w, b2d, dirs_param)


def _sc_agg(fs, fc, dnx_a, dny_a, dnz_a, sx_a, sy_a, sz_a, idxf, oc):
    J = oc // LANES

    @functools.partial(
        pl.kernel,
        out_type=jax.ShapeDtypeStruct((N, oc), jnp.float32),
        mesh=_mesh(),
        compiler_params=pltpu.CompilerParams(needs_layout_passes=False, use_tc_tiling_on_sc=False),
        scratch_types=[
            pltpu.VMEM((oc,), jnp.float32),
            pltpu.VMEM((oc,), jnp.float32),
            pltpu.VMEM((oc,), jnp.float32),
            pltpu.VMEM((VPW * K,), jnp.float32),
            pltpu.VMEM((VPW * K,), jnp.float32),
            pltpu.VMEM((VPW * K,), jnp.float32),
            pltpu.VMEM((VPW * K,), jnp.int32),
            pltpu.VMEM((K, oc), jnp.float32),
            pltpu.VMEM((K, oc), jnp.float32),
            pltpu.VMEM((CH, oc), jnp.float32),
            pltpu.VMEM((CH, oc), jnp.float32),
            pltpu.SemaphoreType.DMA,
            pltpu.SemaphoreType.DMA,
        ],
    )
    def body(fs_hbm, fc_hbm, dnx_hbm, dny_hbm, dnz_hbm, sx_hbm, sy_hbm, sz_hbm,
             idx_hbm, out_hbm,
             sx_v, sy_v, sz_v, dnx_v, dny_v, dnz_v, idx_v,
             rows0, rows1, fc_v, out_v, sem0, sem1):
        wid = lax.axis_index("s") * NC + lax.axis_index("c")
        base = wid * VPW
        pltpu.sync_copy(sx_hbm, sx_v)
        pltpu.sync_copy(sy_hbm, sy_v)
        pltpu.sync_copy(sz_hbm, sz_v)
        pltpu.sync_copy(idx_hbm.at[pl.ds(base * K, VPW * K)], idx_v)
        pltpu.sync_copy(dnx_hbm.at[pl.ds(base * K, VPW * K)], dnx_v)
        pltpu.sync_copy(dny_hbm.at[pl.ds(base * K, VPW * K)], dny_v)
        pltpu.sync_copy(dnz_hbm.at[pl.ds(base * K, VPW * K)], dnz_v)
        rows = (rows0, rows1)
        sems = (sem0, sem1)
        pltpu.async_copy(fs_hbm.at[idx_v.at[pl.ds(0, K)]], rows[0], sems[0])
        pltpu.async_copy(fs_hbm.at[idx_v.at[pl.ds(K, K)]], rows[1], sems[1])

        JJ = min(8, J)
        TILES = J // JJ

        def chunk_body(ch, carry0):
            pltpu.sync_copy(fc_hbm.at[pl.ds(base + ch * CH, CH)], fc_v)

            def pair_body(p, carry):
                for bsel in range(2):
                    v = ch * CH + 2 * p + bsel
                    row = rows[bsel]
                    sem = sems[bsel]
                    pltpu.make_async_copy(
                        fs_hbm.at[idx_v.at[pl.ds(v * K, K)]], row, sem
                    ).wait()
                    vi = 2 * p + bsel
                    dxv = dnx_v[pl.ds(v * K, K)]
                    dyv = dny_v[pl.ds(v * K, K)]
                    dzv = dnz_v[pl.ds(v * K, K)]

                    def tile_body(jt, c, *, row=row, vi=vi,
                                  dxv=dxv, dyv=dyv, dzv=dzv):
                        sls = [pl.ds((jt * JJ + jj) * LANES, LANES)
                               for jj in range(JJ)]
                        sx = [sx_v[sl] for sl in sls]
                        sy = [sy_v[sl] for sl in sls]
                        sz = [sz_v[sl] for sl in sls]
                        acc = [None] * JJ
                        for n in range(K):
                            ax, ay, az = dxv[n], dyv[n], dzv[n]
                            for jj in range(JJ):
                                th = jnp.maximum(
                                    ax * sx[jj] + ay * sy[jj] + az * sz[jj],
                                    0.0,
                                ) * row[n, sls[jj]]
                                acc[jj] = (th if acc[jj] is None
                                           else jnp.maximum(acc[jj], th))
                        for jj in range(JJ):
                            out_v[vi, sls[jj]] = fc_v[vi, sls[jj]] + acc[jj]
                        return c

                    lax.fori_loop(0, TILES, tile_body, 0)
                    @pl.when(v + 2 < VPW)
                    def _():
                        pltpu.async_copy(
                            fs_hbm.at[idx_v.at[pl.ds((v + 2) * K, K)]], row, sem
                        )
                return carry

            lax.fori_loop(0, CH // 2, pair_body, 0)
            pltpu.sync_copy(out_v, out_hbm.at[pl.ds(base + ch * CH, CH)])
            return carry0

        lax.fori_loop(0, VPW // CH, chunk_body, 0)

    return body(fs, fc, dnx_a, dny_a, dnz_a, sx_a, sy_a, sz_a, idxf)


def _head_body(fm_ref, w1_ref, b1_ref, g_ref, bt_ref, w2_ref, b2_ref, out_ref):
    fg = jnp.max(fm_ref[...], axis=1)
    h = jnp.dot(fg, w1_ref[...], preferred_element_type=jnp.float32) + b1_ref[...]
    h = h / jnp.sqrt(jnp.float32(1.0 + 1e-5)) * g_ref[...] + bt_ref[...]
    h = jnp.maximum(h, 0.0)
    out_ref[...] = (
        jnp.dot(h, w2_ref[...], preferred_element_type=jnp.float32) + b2_ref[...]
    )


def _head(fm3, cls):
    c = fm3.shape[-1]
    return pl.pallas_call(
        _head_body,
        out_shape=jax.ShapeDtypeStruct((BS, 2), jnp.float32),
    )(
        fm3,
        cls["w1"],
        cls["b1"].reshape(1, -1),
        cls["gamma"].reshape(1, -1),
        cls["beta"].reshape(1, -1),
        cls["w2"],
        cls["b2"].reshape(1, -1),
    )


def kernel(vertices, params):
    convs = params["convs"]
    cls = params["cls"]
    verts = vertices.reshape(N, 3)
    cx, cy, cz = verts[:, 0], verts[:, 1], verts[:, 2]
    idxf = _knn(vertices).reshape(N * K)
    dx, dy, dz = _sc_dirs(cx, cy, cz, idxf)
    dnx, dny, dnz = _normalize(dx, dy, dz)
    dnx, dny, dnz = dnx.reshape(N * K), dny.reshape(N * K), dnz.reshape(N * K)
    fm = verts
    for p in convs:
        oc = p["weights"].shape[1] // 2
        fc, fs, sdn = _mm(fm, p["weights"], p["bias"].reshape(1, -1), p["directions"])
        fm = _sc_agg(fs, fc, dnx, dny, dnz, sdn[0], sdn[1], sdn[2], idxf, oc)
    return _head(fm.reshape(BS, V, -1), cls)


# speedup vs baseline: 7.5181x; 1.0961x over previous
import functools

import jax
import jax.numpy as jnp
from jax import lax
from jax.experimental import pallas as pl
from jax.experimental.pallas import tpu as pltpu
from jax.experimental.pallas import tpu_sc as plsc

NC, NS, LANES = 2, 16, 16
NW = NC * NS
BS = 2
V = 1024
N = BS * V
K = 16
VPW = N // NW
CH = 16


def _mesh():
    return plsc.VectorSubcoreMesh(
        core_axis_name="c", subcore_axis_name="s", num_cores=NC, num_subcores=NS
    )


def _knn_body(v_ref, idx_ref):
    b = pl.program_id(0)
    vm = v_ref[0]
    inner = lax.dot_general(
        vm, vm, (((1,), (1,)), ((), ())), preferred_element_type=jnp.float32
    )
    q = jnp.sum(vm * vm, axis=1)
    dist = (-2.0 * inner + q[None, :]) + q[:, None]
    col = lax.broadcasted_iota(jnp.int32, (V, V), 1)
    inf = jnp.float32(jnp.inf)
    off = b * V
    outs = []
    for k in range(K + 1):
        m = jnp.min(dist, axis=1)
        cand = jnp.where(dist == m[:, None], col, jnp.int32(2**30))
        amin = jnp.min(cand, axis=1)
        if k > 0:
            outs.append(amin + off)
        dist = jnp.where(col == amin[:, None], inf, dist)
    idx_ref[0] = jnp.stack(outs, axis=-1)


def _knn(vertices):
    return pl.pallas_call(
        _knn_body,
        grid=(BS,),
        in_specs=[pl.BlockSpec((1, V, 3), lambda b: (b, 0, 0))],
        out_specs=pl.BlockSpec((1, V, K), lambda b: (b, 0, 0)),
        out_shape=jax.ShapeDtypeStruct((BS, V, K), jnp.int32),
    )(vertices)


def _sc_dirs(cx, cy, cz, idxf):
    @functools.partial(
        pl.kernel,
        out_type=tuple(jax.ShapeDtypeStruct((N * K,), jnp.float32) for _ in range(3)),
        mesh=_mesh(),
        compiler_params=pltpu.CompilerParams(needs_layout_passes=False, use_tc_tiling_on_sc=False),
        scratch_types=[
            pltpu.VMEM((N,), jnp.float32),
            pltpu.VMEM((N,), jnp.float32),
            pltpu.VMEM((N,), jnp.float32),
            pltpu.VMEM((VPW * K,), jnp.int32),
            pltpu.VMEM((VPW * K,), jnp.float32),
            pltpu.VMEM((VPW * K,), jnp.float32),
            pltpu.VMEM((VPW * K,), jnp.float32),
        ],
    )
    def body(cx_hbm, cy_hbm, cz_hbm, idx_hbm, dx_hbm, dy_hbm, dz_hbm,
             x_v, y_v, z_v, idx_v, dox, doy, doz):
        wid = lax.axis_index("s") * NC + lax.axis_index("c")
        base = wid * VPW
        pltpu.sync_copy(cx_hbm, x_v)
        pltpu.sync_copy(cy_hbm, y_v)
        pltpu.sync_copy(cz_hbm, z_v)
        pltpu.sync_copy(idx_hbm.at[pl.ds(base * K, VPW * K)], idx_v)
        for v in range(VPW):
            iv = idx_v[pl.ds(v * K, K)]
            cidx = jnp.full((LANES,), base + v, jnp.int32)
            for src, dst in ((x_v, dox), (y_v, doy), (z_v, doz)):
                g = plsc.load_gather(src, [iv])
                c = plsc.load_gather(src, [cidx])
                dst[pl.ds(v * K, K)] = g - c
        pltpu.sync_copy(dox, dx_hbm.at[pl.ds(base * K, VPW * K)])
        pltpu.sync_copy(doy, dy_hbm.at[pl.ds(base * K, VPW * K)])
        pltpu.sync_copy(doz, dz_hbm.at[pl.ds(base * K, VPW * K)])

    return body(cx, cy, cz, idxf)


def _norm_body(dx_ref, dy_ref, dz_ref, nx_ref, ny_ref, nz_ref):
    dx, dy, dz = dx_ref[...], dy_ref[...], dz_ref[...]
    n2 = (dx * dx + dy * dy) + dz * dz
    dnm = jnp.maximum(jnp.sqrt(n2), 1e-12)
    nx_ref[...] = dx / dnm
    ny_ref[...] = dy / dnm
    nz_ref[...] = dz / dnm


def _normalize(dx, dy, dz):
    return pl.pallas_call(
        _norm_body,
        out_shape=tuple(jax.ShapeDtypeStruct((N, K), jnp.float32) for _ in range(3)),
    )(dx.reshape(N, K), dy.reshape(N, K), dz.reshape(N, K))


def _mm_body(fm_ref, w_ref, b_ref, dir_ref, fc_ref, fs_ref, sdn_ref, *, oc):
    fo = (
        jnp.dot(fm_ref[...], w_ref[...], preferred_element_type=jnp.float32)
        + b_ref[...]
    )
    fc_ref[...] = fo[:, :oc]
    fs_ref[...] = fo[:, oc:]
    sd = dir_ref[...]
    nrm = jnp.sqrt(jnp.sum(sd * sd, axis=0, keepdims=True))
    sdn_ref[...] = sd / jnp.maximum(nrm, 1e-12)


def _mm(fm, w, b2d, dirs_param):
    ic, oc2 = w.shape
    oc = oc2 // 2
    return pl.pallas_call(
        functools.partial(_mm_body, oc=oc),
        out_shape=(
            jax.ShapeDtypeStruct((N, oc), jnp.float32),
            jax.ShapeDtypeStruct((N, oc), jnp.float32),
            jax.ShapeDtypeStruct((3, oc), jnp.float32),
        ),
    )(fm, w, b2d, dirs_param)


def _sc_agg(fs, fc, dnx_a, dny_a, dnz_a, sx_a, sy_a, sz_a, idxf, oc):
    J = oc // LANES
    GV = min(8, max(1, 2048 // oc))
    G = VPW // GV

    @functools.partial(
        pl.kernel,
        out_type=jax.ShapeDtypeStruct((N, oc), jnp.float32),
        mesh=_mesh(),
        compiler_params=pltpu.CompilerParams(needs_layout_passes=False, use_tc_tiling_on_sc=False),
        scratch_types=[
            pltpu.VMEM((oc,), jnp.float32),
            pltpu.VMEM((oc,), jnp.float32),
            pltpu.VMEM((oc,), jnp.float32),
            pltpu.VMEM((VPW * K,), jnp.float32),
            pltpu.VMEM((VPW * K,), jnp.float32),
            pltpu.VMEM((VPW * K,), jnp.float32),
            pltpu.VMEM((VPW * K,), jnp.int32),
            pltpu.VMEM((GV * K, oc), jnp.float32),
            pltpu.VMEM((GV * K, oc), jnp.float32),
            pltpu.VMEM((CH, oc), jnp.float32),
            pltpu.VMEM((CH, oc), jnp.float32),
            pltpu.SemaphoreType.DMA,
            pltpu.SemaphoreType.DMA,
        ],
    )
    def body(fs_hbm, fc_hbm, dnx_hbm, dny_hbm, dnz_hbm, sx_hbm, sy_hbm, sz_hbm,
             idx_hbm, out_hbm,
             sx_v, sy_v, sz_v, dnx_v, dny_v, dnz_v, idx_v,
             rows0, rows1, fc_v, out_v, sem0, sem1):
        wid = lax.axis_index("s") * NC + lax.axis_index("c")
        base = wid * VPW
        pltpu.sync_copy(idx_hbm.at[pl.ds(base * K, VPW * K)], idx_v)
        rows = (rows0, rows1)
        sems = (sem0, sem1)
        pltpu.async_copy(fs_hbm.at[idx_v.at[pl.ds(0, GV * K)]], rows[0], sems[0])
        pltpu.async_copy(fs_hbm.at[idx_v.at[pl.ds(GV * K, GV * K)]], rows[1], sems[1])
        pltpu.sync_copy(sx_hbm, sx_v)
        pltpu.sync_copy(sy_hbm, sy_v)
        pltpu.sync_copy(sz_hbm, sz_v)
        pltpu.sync_copy(dnx_hbm.at[pl.ds(base * K, VPW * K)], dnx_v)
        pltpu.sync_copy(dny_hbm.at[pl.ds(base * K, VPW * K)], dny_v)
        pltpu.sync_copy(dnz_hbm.at[pl.ds(base * K, VPW * K)], dnz_v)

        JJ = min(8, J)
        TILES = J // JJ
        GPC = CH // GV

        def chunk_body(ch, carry0):
            pltpu.sync_copy(fc_hbm.at[pl.ds(base + ch * CH, CH)], fc_v)

            def pair_body(p, carry):
                for bsel in range(2):
                    g = ch * GPC + 2 * p + bsel
                    row = rows[bsel]
                    sem = sems[bsel]
                    pltpu.make_async_copy(
                        fs_hbm.at[idx_v.at[pl.ds(g * GV * K, GV * K)]], row, sem
                    ).wait()

                    def vert_body(gi, c0):
                        v = g * GV + gi
                        vi = (2 * p + bsel) * GV + gi
                        dxv = dnx_v[pl.ds(v * K, K)]
                        dyv = dny_v[pl.ds(v * K, K)]
                        dzv = dnz_v[pl.ds(v * K, K)]

                        def tile_body(jt, c, *, row=row):
                            sls = [pl.ds((jt * JJ + jj) * LANES, LANES)
                                   for jj in range(JJ)]
                            sx = [sx_v[sl] for sl in sls]
                            sy = [sy_v[sl] for sl in sls]
                            sz = [sz_v[sl] for sl in sls]
                            acc = [None] * JJ
                            for n in range(K):
                                ax, ay, az = dxv[n], dyv[n], dzv[n]
                                for jj in range(JJ):
                                    th = jnp.maximum(
                                        ax * sx[jj] + ay * sy[jj] + az * sz[jj],
                                        0.0,
                                    ) * row[gi * K + n, sls[jj]]
                                    acc[jj] = (th if acc[jj] is None
                                               else jnp.maximum(acc[jj], th))
                            for jj in range(JJ):
                                out_v[vi, sls[jj]] = fc_v[vi, sls[jj]] + acc[jj]
                            return c

                        lax.fori_loop(0, TILES, tile_body, 0)
                        return c0

                    lax.fori_loop(0, GV, vert_body, 0)
                    @pl.when(g + 2 < G)
                    def _():
                        pltpu.async_copy(
                            fs_hbm.at[idx_v.at[pl.ds((g + 2) * GV * K, GV * K)]],
                            row, sem,
                        )
                return carry

            lax.fori_loop(0, GPC // 2, pair_body, 0)
            pltpu.sync_copy(out_v, out_hbm.at[pl.ds(base + ch * CH, CH)])
            return carry0

        lax.fori_loop(0, VPW // CH, chunk_body, 0)

    return body(fs, fc, dnx_a, dny_a, dnz_a, sx_a, sy_a, sz_a, idxf)


def _head_body(fm_ref, w1_ref, b1_ref, g_ref, bt_ref, w2_ref, b2_ref, out_ref):
    fg = jnp.max(fm_ref[...], axis=1)
    h = jnp.dot(fg, w1_ref[...], preferred_element_type=jnp.float32) + b1_ref[...]
    h = h / jnp.sqrt(jnp.float32(1.0 + 1e-5)) * g_ref[...] + bt_ref[...]
    h = jnp.maximum(h, 0.0)
    out_ref[...] = (
        jnp.dot(h, w2_ref[...], preferred_element_type=jnp.float32) + b2_ref[...]
    )


def _head(fm3, cls):
    c = fm3.shape[-1]
    return pl.pallas_call(
        _head_body,
        out_shape=jax.ShapeDtypeStruct((BS, 2), jnp.float32),
    )(
        fm3,
        cls["w1"],
        cls["b1"].reshape(1, -1),
        cls["gamma"].reshape(1, -1),
        cls["beta"].reshape(1, -1),
        cls["w2"],
        cls["b2"].reshape(1, -1),
    )


def kernel(vertices, params):
    convs = params["convs"]
    cls = params["cls"]
    verts = vertices.reshape(N, 3)
    cx, cy, cz = verts[:, 0], verts[:, 1], verts[:, 2]
    idxf = _knn(vertices).reshape(N * K)
    dx, dy, dz = _sc_dirs(cx, cy, cz, idxf)
    dnx, dny, dnz = _normalize(dx, dy, dz)
    dnx, dny, dnz = dnx.reshape(N * K), dny.reshape(N * K), dnz.reshape(N * K)
    fm = verts
    for p in convs:
        oc = p["weights"].shape[1] // 2
        fc, fs, sdn = _mm(fm, p["weights"], p["bias"].reshape(1, -1), p["directions"])
        fm = _sc_agg(fs, fc, dnx, dny, dnz, sdn[0], sdn[1], sdn[2], idxf, oc)
    return _head(fm.reshape(BS, V, -1), cls)
